# baseline (speedup 1.0000x reference)
import jax
import jax.numpy as jnp
from jax.experimental import pallas as pl

_EMB = 64
_DIN = 32
_NROWS = 4


def _fused_kernel(seg_ref, x_ref, w_ref, b_ref, table_ref, o_ref):
    x = x_ref[...]
    w = w_ref[...]
    bias = b_ref[...]
    table = table_ref[...]
    seg = seg_ref[...]

    onehot = (seg == jax.lax.broadcasted_iota(jnp.int32, (1, _NROWS), 1)
              ).astype(jnp.float32)
    emb = jnp.dot(onehot, table, preferred_element_type=jnp.float32)

    dense = jax.lax.dot_general(
        x, w, (((2,), (0,)), ((), ())),
        preferred_element_type=jnp.float32)
    o_ref[...] = dense + (emb + bias)[None, :, :]


@jax.jit
def kernel(x, W, b, table, seg):
    B, P, DIN = x.shape
    BT = 8
    seg2d = seg.reshape(P, 1)
    b2d = b.reshape(1, _EMB)

    grid = (B // BT,)
    return pl.pallas_call(
        _fused_kernel,
        grid=grid,
        in_specs=[
            pl.BlockSpec((P, 1), lambda i: (0, 0)),
            pl.BlockSpec((BT, P, DIN), lambda i: (i, 0, 0)),
            pl.BlockSpec((DIN, _EMB), lambda i: (0, 0)),
            pl.BlockSpec((1, _EMB), lambda i: (0, 0)),
            pl.BlockSpec((_NROWS, _EMB), lambda i: (0, 0)),
        ],
        out_specs=pl.BlockSpec((BT, P, _EMB), lambda i: (i, 0, 0)),
        out_shape=jax.ShapeDtypeStruct((B, P, _EMB), jnp.float32),
    )(seg2d, x, W, b2d, table)


# speedup vs baseline: 2.2752x; 2.2752x over previous
import jax
import jax.numpy as jnp
from jax.experimental import pallas as pl

_EMB = 64
_DIN = 32
_NROWS = 4
_FOLD = 4


def _fused_kernel(seg_ref, x_ref, w_ref, b_ref, table_ref, o_ref):
    x = x_ref[...]
    w = w_ref[...]
    bias = b_ref[...]
    table = table_ref[...]
    seg = seg_ref[...]

    rvec = jax.lax.broadcasted_iota(
        jnp.int32, (1, _FOLD * _NROWS), 1) % _NROWS
    onehot = (seg == rvec).astype(jnp.float32)
    emb = jnp.dot(onehot, table, preferred_element_type=jnp.float32)

    dense = jax.lax.dot_general(
        x, w, (((2,), (0,)), ((), ())),
        preferred_element_type=jnp.float32)
    o_ref[...] = dense + (emb + bias)[None, :, :]


@jax.jit
def kernel(x, W, b, table, seg):
    B, P, DIN = x.shape
    P4 = P // _FOLD
    BT = 16

    x4 = x.reshape(B, P4, _FOLD * DIN)
    eye = jnp.eye(_FOLD, dtype=jnp.float32)
    wbig = jnp.kron(eye, W)
    tbig = jnp.kron(eye, table)
    b4 = jnp.tile(b, _FOLD).reshape(1, _FOLD * _EMB)
    segrep = jnp.repeat(seg.reshape(P4, _FOLD), _NROWS, axis=1)

    grid = (B // BT,)
    out4 = pl.pallas_call(
        _fused_kernel,
        grid=grid,
        in_specs=[
            pl.BlockSpec((P4, _FOLD * _NROWS), lambda i: (0, 0)),
            pl.BlockSpec((BT, P4, _FOLD * DIN), lambda i: (i, 0, 0)),
            pl.BlockSpec((_FOLD * DIN, _FOLD * _EMB), lambda i: (0, 0)),
            pl.BlockSpec((1, _FOLD * _EMB), lambda i: (0, 0)),
            pl.BlockSpec((_FOLD * _NROWS, _FOLD * _EMB), lambda i: (0, 0)),
        ],
        out_specs=pl.BlockSpec((BT, P4, _FOLD * _EMB), lambda i: (i, 0, 0)),
        out_shape=jax.ShapeDtypeStruct((B, P4, _FOLD * _EMB), jnp.float32),
    )(segrep, x4, wbig, b4, tbig)
    return out4.reshape(B, P, _EMB)
